# baseline (device time: 11112 ns/iter reference)
import jax
import jax.numpy as jnp
from jax import lax
from jax.experimental import pallas as pl
from jax.experimental.pallas import tpu as pltpu

N_DEV = 4
BLOCK_ORDER = (2, 1, 3, 0)


def _gelu(y):
    c = 0.7978845608028654
    return 0.5 * y * (1.0 + jnp.tanh(c * (y + 0.044715 * y * y * y)))


def kernel(x, w_mat):
    m_per, k = x.shape
    n = w_mat.shape[1]
    n_per = n // N_DEV

    def body(x_hbm, w_hbm, out_ref, xv, wv, zs,
             load_sems, send_sems, recv_sems):
        my = lax.axis_index("i")

        barrier_sem = pltpu.get_barrier_semaphore()
        for off in range(1, N_DEV):
            peer = (my + off) % N_DEV
            pl.semaphore_signal(
                barrier_sem, inc=1,
                device_id=(peer,), device_id_type=pl.DeviceIdType.MESH,
            )

        cp_x = pltpu.make_async_copy(x_hbm, xv, load_sems.at[N_DEV])
        cp_x.start()
        wcps = []
        for i, off in enumerate(BLOCK_ORDER):
            tgt = (my + off) % N_DEV
            cp = pltpu.make_async_copy(
                w_hbm.at[:, pl.ds(tgt * n_per, n_per)],
                wv.at[i],
                load_sems.at[i],
            )
            cp.start()
            wcps.append(cp)

        cp_x.wait()
        xb = xv[...].astype(jnp.bfloat16)

        for i, off in enumerate(BLOCK_ORDER):
            wcps[i].wait()
            wb = wv[i].astype(jnp.bfloat16)
            blk = _gelu(jnp.dot(xb, wb, preferred_element_type=jnp.float32))
            if off == 0:
                out_ref[pl.ds(my * m_per, m_per), :] = blk.astype(jnp.bfloat16)
            else:
                zs[off - 1] = blk.astype(jnp.bfloat16)
        pl.semaphore_wait(barrier_sem, N_DEV - 1)
        rdmas = []
        for off in (2, 1, 3):
            rdma = pltpu.make_async_remote_copy(
                src_ref=zs.at[off - 1],
                dst_ref=out_ref.at[pl.ds(my * m_per, m_per), :],
                send_sem=send_sems.at[off - 1],
                recv_sem=recv_sems.at[off - 1],
                device_id=((my + off) % N_DEV,),
                device_id_type=pl.DeviceIdType.MESH,
            )
            rdma.start()
            rdmas.append(rdma)

        for off in range(1, N_DEV):
            src = (my - off) % N_DEV
            recv = pltpu.make_async_remote_copy(
                src_ref=zs.at[off - 1],
                dst_ref=out_ref.at[pl.ds(src * m_per, m_per), :],
                send_sem=send_sems.at[off - 1],
                recv_sem=recv_sems.at[off - 1],
                device_id=(src,),
                device_id_type=pl.DeviceIdType.MESH,
            )
            recv.wait_recv()
        for rdma in rdmas:
            rdma.wait_send()

    out_shape = jax.ShapeDtypeStruct((N_DEV * m_per, n_per), jnp.bfloat16)
    run = pl.pallas_call(
        body,
        out_shape=out_shape,
        in_specs=[
            pl.BlockSpec(memory_space=pltpu.MemorySpace.HBM),
            pl.BlockSpec(memory_space=pltpu.MemorySpace.HBM),
        ],
        out_specs=pl.BlockSpec(memory_space=pltpu.VMEM),
        scratch_shapes=[
            pltpu.VMEM((m_per, k), jnp.float32),
            pltpu.VMEM((N_DEV, k, n_per), jnp.float32),
            pltpu.VMEM((N_DEV - 1, m_per, n_per), jnp.bfloat16),
            pltpu.SemaphoreType.DMA((N_DEV + 1,)),
            pltpu.SemaphoreType.DMA((N_DEV - 1,)),
            pltpu.SemaphoreType.DMA((N_DEV - 1,)),
        ],
        compiler_params=pltpu.CompilerParams(collective_id=0),
    )
    return run(
        pltpu.with_memory_space_constraint(x, pltpu.MemorySpace.HBM),
        pltpu.with_memory_space_constraint(w_mat, pltpu.MemorySpace.HBM),
    )


# device time: 10371 ns/iter; 1.0714x vs baseline; 1.0714x over previous
import jax
import jax.numpy as jnp
from jax import lax
from jax.experimental import pallas as pl
from jax.experimental.pallas import tpu as pltpu

N_DEV = 4
BLOCK_ORDER = (2, 1, 3, 0)


def _gelu(y):
    c = 0.7978845608028654
    return 0.5 * y * (1.0 + jnp.tanh(c * (y + 0.044715 * y * y * y)))


def kernel(x, w_mat):
    m_per, k = x.shape
    n = w_mat.shape[1]
    n_per = n // N_DEV

    def body(x_hbm, w_hbm, out_ref, xv, wv, zs,
             load_sems, send_sems, recv_sems):
        my = lax.axis_index("i")

        barrier_sem = pltpu.get_barrier_semaphore()
        for off in range(1, N_DEV):
            peer = (my + off) % N_DEV
            pl.semaphore_signal(
                barrier_sem, inc=1,
                device_id=(peer,), device_id_type=pl.DeviceIdType.MESH,
            )

        cp_x = pltpu.make_async_copy(x_hbm, xv, load_sems.at[N_DEV])
        cp_x.start()
        wcps = []
        for i, off in enumerate(BLOCK_ORDER):
            tgt = (my + off) % N_DEV
            cp = pltpu.make_async_copy(
                w_hbm.at[:, pl.ds(tgt * n_per, n_per)],
                wv.at[i],
                load_sems.at[i],
            )
            cp.start()
            wcps.append(cp)

        cp_x.wait()
        xb = xv[...].astype(jnp.bfloat16)

        rdmas = []
        for i, off in enumerate(BLOCK_ORDER):
            wcps[i].wait()
            wb = wv[i].astype(jnp.bfloat16)
            blk = _gelu(jnp.dot(xb, wb, preferred_element_type=jnp.float32))
            if off == 0:
                out_ref[pl.ds(my * m_per, m_per), :] = blk.astype(jnp.bfloat16)
                continue
            zs[off - 1] = blk.astype(jnp.bfloat16)
            if i == 0:
                pl.semaphore_wait(barrier_sem, N_DEV - 1)
            rdma = pltpu.make_async_remote_copy(
                src_ref=zs.at[off - 1],
                dst_ref=out_ref.at[pl.ds(my * m_per, m_per), :],
                send_sem=send_sems.at[off - 1],
                recv_sem=recv_sems.at[off - 1],
                device_id=((my + off) % N_DEV,),
                device_id_type=pl.DeviceIdType.MESH,
            )
            rdma.start()
            rdmas.append(rdma)

        for off in range(1, N_DEV):
            src = (my - off) % N_DEV
            recv = pltpu.make_async_remote_copy(
                src_ref=zs.at[off - 1],
                dst_ref=out_ref.at[pl.ds(src * m_per, m_per), :],
                send_sem=send_sems.at[off - 1],
                recv_sem=recv_sems.at[off - 1],
                device_id=(src,),
                device_id_type=pl.DeviceIdType.MESH,
            )
            recv.wait_recv()
        for rdma in rdmas:
            rdma.wait_send()

    out_shape = jax.ShapeDtypeStruct((N_DEV * m_per, n_per), jnp.bfloat16)
    run = pl.pallas_call(
        body,
        out_shape=out_shape,
        in_specs=[
            pl.BlockSpec(memory_space=pltpu.MemorySpace.HBM),
            pl.BlockSpec(memory_space=pltpu.MemorySpace.HBM),
        ],
        out_specs=pl.BlockSpec(memory_space=pltpu.VMEM),
        scratch_shapes=[
            pltpu.VMEM((m_per, k), jnp.float32),
            pltpu.VMEM((N_DEV, k, n_per), jnp.float32),
            pltpu.VMEM((N_DEV - 1, m_per, n_per), jnp.bfloat16),
            pltpu.SemaphoreType.DMA((N_DEV + 1,)),
            pltpu.SemaphoreType.DMA((N_DEV - 1,)),
            pltpu.SemaphoreType.DMA((N_DEV - 1,)),
        ],
        compiler_params=pltpu.CompilerParams(collective_id=0),
    )
    return run(
        pltpu.with_memory_space_constraint(x, pltpu.MemorySpace.HBM),
        pltpu.with_memory_space_constraint(w_mat, pltpu.MemorySpace.HBM),
    )


# device time: 10317 ns/iter; 1.0771x vs baseline; 1.0052x over previous
import jax
import jax.numpy as jnp
from jax import lax
from jax.experimental import pallas as pl
from jax.experimental.pallas import tpu as pltpu

N_DEV = 4
BLOCK_ORDER = (2, 1, 3, 0)


def _gelu(y):
    c = 0.7978845608028654
    return 0.5 * y * (1.0 + jnp.tanh(c * (y + 0.044715 * y * y * y)))


def kernel(x, w_mat):
    m_per, k = x.shape
    n = w_mat.shape[1]
    n_per = n // N_DEV
    m_half = m_per // 2

    def body(x_hbm, w_hbm, out_ref, xv, wv, zs,
             load_sems, send_sems, recv_sems):
        my = lax.axis_index("i")

        barrier_sem = pltpu.get_barrier_semaphore()
        for off in range(1, N_DEV):
            peer = (my + off) % N_DEV
            pl.semaphore_signal(
                barrier_sem, inc=1,
                device_id=(peer,), device_id_type=pl.DeviceIdType.MESH,
            )

        cp_x0 = pltpu.make_async_copy(
            x_hbm.at[pl.ds(0, m_half), :], xv.at[pl.ds(0, m_half), :],
            load_sems.at[N_DEV],
        )
        cp_x1 = pltpu.make_async_copy(
            x_hbm.at[pl.ds(m_half, m_half), :], xv.at[pl.ds(m_half, m_half), :],
            load_sems.at[N_DEV + 1],
        )
        cp_x0.start()
        cp_x1.start()
        wcps = []
        for i, off in enumerate(BLOCK_ORDER):
            tgt = (my + off) % N_DEV
            cp = pltpu.make_async_copy(
                w_hbm.at[:, pl.ds(tgt * n_per, n_per)],
                wv.at[i],
                load_sems.at[i],
            )
            cp.start()
            wcps.append(cp)

        wcps[0].wait()
        wb_diag = wv[0].astype(jnp.bfloat16)
        cp_x0.wait()
        xb0 = xv[pl.ds(0, m_half), :].astype(jnp.bfloat16)
        blk = _gelu(jnp.dot(xb0, wb_diag, preferred_element_type=jnp.float32))
        zs[1, pl.ds(0, m_half), :] = blk.astype(jnp.bfloat16)
        pl.semaphore_wait(barrier_sem, N_DEV - 1)
        diag = (my + 2) % N_DEV
        rdma_d0 = pltpu.make_async_remote_copy(
            src_ref=zs.at[1, pl.ds(0, m_half), :],
            dst_ref=out_ref.at[pl.ds(my * m_per, m_half), :],
            send_sem=send_sems.at[1],
            recv_sem=recv_sems.at[1],
            device_id=(diag,),
            device_id_type=pl.DeviceIdType.MESH,
        )
        rdma_d0.start()
        cp_x1.wait()
        xb1 = xv[pl.ds(m_half, m_half), :].astype(jnp.bfloat16)
        blk = _gelu(jnp.dot(xb1, wb_diag, preferred_element_type=jnp.float32))
        zs[1, pl.ds(m_half, m_half), :] = blk.astype(jnp.bfloat16)
        rdma_d1 = pltpu.make_async_remote_copy(
            src_ref=zs.at[1, pl.ds(m_half, m_half), :],
            dst_ref=out_ref.at[pl.ds(my * m_per + m_half, m_half), :],
            send_sem=send_sems.at[N_DEV - 1],
            recv_sem=recv_sems.at[N_DEV - 1],
            device_id=(diag,),
            device_id_type=pl.DeviceIdType.MESH,
        )
        rdma_d1.start()
        rdmas = [rdma_d0, rdma_d1]

        xb = jnp.concatenate([xb0, xb1], axis=0)

        for i, off in enumerate(BLOCK_ORDER):
            if off == 2:
                continue
            wcps[i].wait()
            wb = wv[i].astype(jnp.bfloat16)
            blk = _gelu(jnp.dot(xb, wb, preferred_element_type=jnp.float32))
            if off == 0:
                out_ref[pl.ds(my * m_per, m_per), :] = blk.astype(jnp.bfloat16)
                continue
            zs[off - 1] = blk.astype(jnp.bfloat16)
            rdma = pltpu.make_async_remote_copy(
                src_ref=zs.at[off - 1],
                dst_ref=out_ref.at[pl.ds(my * m_per, m_per), :],
                send_sem=send_sems.at[off - 1],
                recv_sem=recv_sems.at[off - 1],
                device_id=((my + off) % N_DEV,),
                device_id_type=pl.DeviceIdType.MESH,
            )
            rdma.start()
            rdmas.append(rdma)

        for off in range(1, N_DEV):
            src = (my - off) % N_DEV
            if off == 2:
                parts = [
                    (pl.ds(src * m_per, m_half), 1),
                    (pl.ds(src * m_per + m_half, m_half), N_DEV - 1),
                ]
            else:
                parts = [(pl.ds(src * m_per, m_per), off - 1)]
            for rows, slot in parts:
                recv = pltpu.make_async_remote_copy(
                    src_ref=zs.at[0],
                    dst_ref=out_ref.at[rows, :],
                    send_sem=send_sems.at[slot],
                    recv_sem=recv_sems.at[slot],
                    device_id=(src,),
                    device_id_type=pl.DeviceIdType.MESH,
                )
                recv.wait_recv()
        for rdma in rdmas:
            rdma.wait_send()

    out_shape = jax.ShapeDtypeStruct((N_DEV * m_per, n_per), jnp.bfloat16)
    run = pl.pallas_call(
        body,
        out_shape=out_shape,
        in_specs=[
            pl.BlockSpec(memory_space=pltpu.MemorySpace.HBM),
            pl.BlockSpec(memory_space=pltpu.MemorySpace.HBM),
        ],
        out_specs=pl.BlockSpec(memory_space=pltpu.VMEM),
        scratch_shapes=[
            pltpu.VMEM((m_per, k), jnp.float32),
            pltpu.VMEM((N_DEV, k, n_per), jnp.float32),
            pltpu.VMEM((N_DEV - 1, m_per, n_per), jnp.bfloat16),
            pltpu.SemaphoreType.DMA((N_DEV + 2,)),
            pltpu.SemaphoreType.DMA((N_DEV,)),
            pltpu.SemaphoreType.DMA((N_DEV,)),
        ],
        compiler_params=pltpu.CompilerParams(collective_id=0),
    )
    return run(
        pltpu.with_memory_space_constraint(x, pltpu.MemorySpace.HBM),
        pltpu.with_memory_space_constraint(w_mat, pltpu.MemorySpace.HBM),
    )
